# baseline (device time: 14463 ns/iter reference)
import jax
import jax.numpy as jnp
from jax import lax
from jax.experimental import pallas as pl
from jax.experimental.pallas import tpu as pltpu

N_DEV = 8
N_CHUNKS = 8


def kernel(x):
    m, n = x.shape
    ch = m // N_CHUNKS

    def body(x_ref, out_ref, buf_ref, comm_ref, copy_sems, send_sems, recv_sems):
        my = lax.axis_index("i")

        barrier_sem = pltpu.get_barrier_semaphore()
        for o in range(1, N_DEV):
            pl.semaphore_signal(
                barrier_sem, inc=1,
                device_id=((my + o) % N_DEV,),
                device_id_type=pl.DeviceIdType.MESH,
            )

        copies = []
        for k in range(N_CHUNKS):
            cp = pltpu.make_async_copy(
                x_ref.at[pl.ds(k * ch, ch), :],
                buf_ref.at[k % 2],
                copy_sems.at[k % 2],
            )
            copies.append(cp)
        copies[0].start()
        acc = None
        for k in range(N_CHUNKS):
            copies[k].wait()
            if k + 1 < N_CHUNKS:
                copies[k + 1].start()
            part = jnp.max(buf_ref[k % 2], axis=0, keepdims=True)
            acc = part if acc is None else jnp.maximum(acc, part)
        comm_ref[0, :, :] = acc

        pl.semaphore_wait(barrier_sem, N_DEV - 1)

        rdmas = []
        for o in range(1, N_DEV):
            rdma = pltpu.make_async_remote_copy(
                src_ref=comm_ref.at[0],
                dst_ref=comm_ref.at[o],
                send_sem=send_sems.at[o - 1],
                recv_sem=recv_sems.at[o - 1],
                device_id=((my + o) % N_DEV,),
                device_id_type=pl.DeviceIdType.MESH,
            )
            rdma.start()
            rdmas.append(rdma)

        for o in range(1, N_DEV):
            rdmas[o - 1].wait_recv()
            acc = jnp.maximum(acc, comm_ref[o, :, :])
        out_ref[...] = acc

        for o in range(1, N_DEV):
            rdmas[o - 1].wait_send()

    return pl.pallas_call(
        body,
        out_shape=jax.ShapeDtypeStruct((1, n), x.dtype),
        in_specs=[pl.BlockSpec(memory_space=pl.ANY)],
        out_specs=pl.BlockSpec(memory_space=pltpu.VMEM),
        scratch_shapes=[
            pltpu.VMEM((2, ch, n), x.dtype),
            pltpu.VMEM((N_DEV, 1, n), x.dtype),
            pltpu.SemaphoreType.DMA((2,)),
            pltpu.SemaphoreType.DMA((N_DEV - 1,)),
            pltpu.SemaphoreType.DMA((N_DEV - 1,)),
        ],
        compiler_params=pltpu.CompilerParams(collective_id=0),
    )(x)


# device time: 14392 ns/iter; 1.0049x vs baseline; 1.0049x over previous
import jax
import jax.numpy as jnp
from jax import lax
from jax.experimental import pallas as pl
from jax.experimental.pallas import tpu as pltpu

N_DEV = 8
N_CHUNKS = 8


def kernel(x):
    m, n = x.shape
    ch = m // N_CHUNKS

    def body(x_ref, out_ref, buf_ref, comm_ref, copy_sems, send_sems, recv_sems):
        my = lax.axis_index("i")

        barrier_sem = pltpu.get_barrier_semaphore()
        for o in range(1, N_DEV):
            pl.semaphore_signal(
                barrier_sem, inc=1,
                device_id=((my + o) % N_DEV,),
                device_id_type=pl.DeviceIdType.MESH,
            )

        copies = []
        for k in range(N_CHUNKS):
            cp = pltpu.make_async_copy(
                x_ref.at[pl.ds(k * ch, ch), :],
                buf_ref.at[k],
                copy_sems.at[k],
            )
            cp.start()
            copies.append(cp)
        acc = None
        for k in range(N_CHUNKS):
            copies[k].wait()
            part = jnp.max(buf_ref[k], axis=0, keepdims=True)
            acc = part if acc is None else jnp.maximum(acc, part)
        comm_ref[0, :, :] = acc

        pl.semaphore_wait(barrier_sem, N_DEV - 1)

        rdmas = []
        for o in range(1, N_DEV):
            rdma = pltpu.make_async_remote_copy(
                src_ref=comm_ref.at[0],
                dst_ref=comm_ref.at[o],
                send_sem=send_sems.at[o - 1],
                recv_sem=recv_sems.at[o - 1],
                device_id=((my + o) % N_DEV,),
                device_id_type=pl.DeviceIdType.MESH,
            )
            rdma.start()
            rdmas.append(rdma)

        for o in range(1, N_DEV):
            rdmas[o - 1].wait_recv()
            acc = jnp.maximum(acc, comm_ref[o, :, :])
        out_ref[...] = acc

        for o in range(1, N_DEV):
            rdmas[o - 1].wait_send()

    return pl.pallas_call(
        body,
        out_shape=jax.ShapeDtypeStruct((1, n), x.dtype),
        in_specs=[pl.BlockSpec(memory_space=pl.ANY)],
        out_specs=pl.BlockSpec(memory_space=pltpu.VMEM),
        scratch_shapes=[
            pltpu.VMEM((N_CHUNKS, ch, n), x.dtype),
            pltpu.VMEM((N_DEV, 1, n), x.dtype),
            pltpu.SemaphoreType.DMA((N_CHUNKS,)),
            pltpu.SemaphoreType.DMA((N_DEV - 1,)),
            pltpu.SemaphoreType.DMA((N_DEV - 1,)),
        ],
        compiler_params=pltpu.CompilerParams(collective_id=0),
    )(x)
